# baseline (speedup 1.0000x reference)
import jax, jax.numpy as jnp
from jax.experimental import pallas as pl

def _noop_pallas(x):
    def k(x_ref, o_ref):
        o_ref[...] = x_ref[...]
    return pl.pallas_call(k, out_shape=jax.ShapeDtypeStruct(x.shape, x.dtype))(x)

def kernel(x, edge_index, edge_f, edge_attr, lin1_W, lin1_b, lin2_W, lin2_b, linw_W, linw_b,
           gcn_W, gcn_b, bn_e_gamma, bn_e_beta, bn_n_gamma, bn_n_beta):
    n = x.shape[0]
    src = edge_index[0]; dst = edge_index[1]
    h = x @ lin1_W.T + lin1_b
    x_em = jax.nn.relu(h[src] + h[dst])
    msg = jnp.concatenate([x_em, edge_attr, edge_f], axis=-1)
    ef = msg @ lin2_W.T + lin2_b
    mu_e = jnp.mean(ef, axis=0)
    var_e = jnp.mean((ef - mu_e) ** 2, axis=0)
    ef = (ef - mu_e) / jnp.sqrt(var_e + 1e-5) * bn_e_gamma + bn_e_beta
    ef = jax.nn.relu(ef)
    ew = jax.nn.relu(ef @ linw_W.T + linw_b)[:, 0]
    loop = jnp.arange(n, dtype=src.dtype)
    row = jnp.concatenate([src, loop]); col = jnp.concatenate([dst, loop])
    w = jnp.concatenate([ew, jnp.ones((n,), dtype=ew.dtype)])
    deg = jax.ops.segment_sum(w, col, num_segments=n)
    dinv = jnp.where(deg > 0, 1.0 / jnp.sqrt(deg), 0.0)
    norm = dinv[row] * w * dinv[col]
    xw = x @ gcn_W.T
    agg = jnp.zeros((n, gcn_W.shape[0]), dtype=x.dtype).at[col].add(norm[:, None] * xw[row])
    agg = agg + gcn_b
    mu_n = jnp.mean(agg, axis=0)
    var_n = jnp.mean((agg - mu_n) ** 2, axis=0)
    x1 = (agg - mu_n) / jnp.sqrt(var_n + 1e-5) * bn_n_gamma + bn_n_beta
    x1 = jax.nn.relu(x1)
    return (_noop_pallas(x1 + x), ef + edge_attr)


# speedup vs baseline: 7.5191x; 7.5191x over previous
import functools

import jax
import jax.numpy as jnp
from jax import lax
from jax.experimental import pallas as pl
from jax.experimental.pallas import tpu as pltpu
from jax.experimental.pallas import tpu_sc as plsc

_N = 10000
_E = 320000
_D = 128
_DE = 16
_NC = 2
_NS = 16
_NW = _NC * _NS
_C = 80
_EPW = _E // _NW
_NCH = _EPW // _C
_NPAD = 10240
_ZPT = _NPAD // _NS
_RPT = _NPAD // _NS
_EB = 16000
_GE = _E // _EB


def _dot_t(a, b):
    return lax.dot_general(a, b, (((1,), (1,)), ((), ())),
                           preferred_element_type=jnp.float32)


def _k1_body(x_ref, w1_ref, wg_ref, b1_ref, h_ref, xw_ref):
    xv = x_ref[...]
    h_ref[...] = _dot_t(xv, w1_ref[...]) + b1_ref[...]
    xw_ref[...] = _dot_t(xv, wg_ref[...])


def _dense_pre(x, lin1_W, gcn_W, lin1_b):
    return pl.pallas_call(
        _k1_body,
        out_shape=(jax.ShapeDtypeStruct((_N, _D), jnp.float32),
                   jax.ShapeDtypeStruct((_N, _D), jnp.float32)),
    )(x, lin1_W, gcn_W, lin1_b)


def _sc_mesh():
    return plsc.VectorSubcoreMesh(core_axis_name="c", subcore_axis_name="s")


def _sc_gather_relu(h, src3, dst3):
    @functools.partial(
        pl.kernel,
        out_type=jax.ShapeDtypeStruct((_E, _D), jnp.float32),
        mesh=_sc_mesh(),
        scratch_types=[
            pltpu.VMEM((_NCH, _C), jnp.int32),
            pltpu.VMEM((_NCH, _C), jnp.int32),
            pltpu.VMEM((_C, _D), jnp.float32),
            pltpu.VMEM((_C, _D), jnp.float32),
            pltpu.SemaphoreType.DMA,
            pltpu.SemaphoreType.DMA,
        ],
    )
    def k(h_hbm, src_hbm, dst_hbm, s_hbm, sidx, didx, b1, b2, sem1, sem2):
        wid = lax.axis_index("c") * _NS + lax.axis_index("s")
        pltpu.sync_copy(src_hbm.at[wid], sidx)
        pltpu.sync_copy(dst_hbm.at[wid], didx)
        ebase = pl.multiple_of(wid * _EPW, _C)

        def chunk(i, carry):
            c1 = pltpu.async_copy(h_hbm.at[sidx.at[i]], b1, sem1)
            c2 = pltpu.async_copy(h_hbm.at[didx.at[i]], b2, sem2)
            c1.wait()
            c2.wait()

            def row(r, carry2):
                for j in range(_D // 16):
                    sl = pl.ds(j * 16, 16)
                    b1[r, sl] = jnp.maximum(b1[r, sl] + b2[r, sl], 0.0)
                return carry2

            lax.fori_loop(0, _C, row, 0)
            pltpu.sync_copy(b1, s_hbm.at[pl.ds(ebase + i * _C, _C)])
            return carry

        lax.fori_loop(0, _NCH, chunk, 0)

    return k(h, src3, dst3)


def _k4_body(s_ref, ea_ref, efe_ref, wem_ref, wea_ref, wef_ref,
             raw_ref, st_ref):
    i = pl.program_id(0)
    raw = (_dot_t(s_ref[...], wem_ref[...])
           + _dot_t(ea_ref[...], wea_ref[...])
           + _dot_t(efe_ref[...], wef_ref[...]))
    raw_ref[...] = raw

    @pl.when(i == 0)
    def _():
        st_ref[...] = jnp.zeros_like(st_ref)

    st_ref[0, :] += jnp.sum(raw, axis=0)
    st_ref[1, :] += jnp.sum(raw * raw, axis=0)


_EB4 = 4000
_GE4 = _E // _EB4


def _edge_mlp(s, edge_attr, edge_f, wem, wea, wef):
    return pl.pallas_call(
        _k4_body,
        grid=(_GE4,),
        in_specs=[
            pl.BlockSpec((_EB4, _D), lambda i: (i, 0)),
            pl.BlockSpec((_EB4, _DE), lambda i: (i, 0)),
            pl.BlockSpec((_EB4, _DE), lambda i: (i, 0)),
            pl.BlockSpec((_DE, _D), lambda i: (0, 0)),
            pl.BlockSpec((_DE, _DE), lambda i: (0, 0)),
            pl.BlockSpec((_DE, _DE), lambda i: (0, 0)),
        ],
        out_specs=(pl.BlockSpec((_EB4, _DE), lambda i: (i, 0)),
                   pl.BlockSpec((2, _DE), lambda i: (0, 0))),
        out_shape=(jax.ShapeDtypeStruct((_E, _DE), jnp.float32),
                   jax.ShapeDtypeStruct((2, _DE), jnp.float32)),
    )(s, edge_attr, edge_f, wem, wea, wef)


def _k5_body(raw_ref, st_ref, g_ref, b_ref, lw_ref, lwb_ref, ea_ref,
             selb_ref, oef_ref, ew_ref):
    mu = st_ref[0, :] / _E
    var = st_ref[1, :] / _E - mu * mu
    rstd = lax.rsqrt(var + 1e-5)
    sc16 = rstd * g_ref[...]
    sh16 = b_ref[...] - mu * sc16
    sc128 = jnp.tile(sc16, 8)
    sh128 = jnp.tile(sh16, 8)
    ef = jnp.maximum(raw_ref[...] * sc128 + sh128, 0.0)
    oef_ref[...] = ef + ea_ref[...]
    w128 = jnp.tile(lw_ref[0, :], 8)
    ewv = lax.dot_general(ef * w128, selb_ref[...], (((1,), (0,)), ((), ())),
                          preferred_element_type=jnp.float32) + lwb_ref[0]
    ew_ref[...] = jnp.maximum(ewv, 0.0)


def _edge_bn(raw2, stats, gamma, beta, linw_W, linw_b, ea2, selb):
    e8 = _E // 8
    eb8 = _EB // 8
    return pl.pallas_call(
        _k5_body,
        grid=(_GE,),
        in_specs=[
            pl.BlockSpec((eb8, 128), lambda i: (i, 0)),
            pl.BlockSpec((2, _DE), lambda i: (0, 0)),
            pl.BlockSpec((_DE,), lambda i: (0,)),
            pl.BlockSpec((_DE,), lambda i: (0,)),
            pl.BlockSpec((1, _DE), lambda i: (0, 0)),
            pl.BlockSpec((1,), lambda i: (0,)),
            pl.BlockSpec((eb8, 128), lambda i: (i, 0)),
            pl.BlockSpec((128, 8), lambda i: (0, 0)),
        ],
        out_specs=(pl.BlockSpec((eb8, 128), lambda i: (i, 0)),
                   pl.BlockSpec((eb8, 8), lambda i: (i, 0))),
        out_shape=(jax.ShapeDtypeStruct((e8, 128), jnp.float32),
                   jax.ShapeDtypeStruct((e8, 8), jnp.float32)),
    )(raw2, stats, gamma, beta, linw_W, linw_b, ea2, selb)


def _sc_deg(dst3, ew3):
    @functools.partial(
        pl.kernel,
        out_type=jax.ShapeDtypeStruct((_NC, _NPAD), jnp.float32),
        mesh=_sc_mesh(),
        scratch_types=[
            pltpu.VMEM((_NCH, _C), jnp.int32),
            pltpu.VMEM((_NCH, _C), jnp.float32),
            pltpu.VMEM((_ZPT,), jnp.float32),
            pltpu.VMEM_SHARED((_NPAD,), jnp.float32),
        ],
    )
    def k(dst_hbm, ew_hbm, out_hbm, didx, ewv, zb, degs):
        cid = lax.axis_index("c")
        sid = lax.axis_index("s")
        wid = cid * _NS + sid
        pltpu.sync_copy(dst_hbm.at[wid], didx)
        pltpu.sync_copy(ew_hbm.at[wid], ewv)
        for j in range(_ZPT // 16):
            zb[pl.ds(j * 16, 16)] = jnp.zeros((16,), jnp.float32)
        zbase = pl.multiple_of(sid * _ZPT, 8)
        pltpu.sync_copy(zb, degs.at[pl.ds(zbase, _ZPT)])
        plsc.subcore_barrier()

        def chunk(i, carry):
            pltpu.sync_copy(ewv.at[i], degs.at[didx.at[i]], add=True)
            return carry

        lax.fori_loop(0, _NCH, chunk, 0)
        plsc.subcore_barrier()
        pltpu.sync_copy(degs.at[pl.ds(zbase, _ZPT)],
                        out_hbm.at[cid, pl.ds(zbase, _ZPT)])

    return k(dst3, ew3)


def _k7_body(dp_ref, dinv_ref):
    deg = dp_ref[0] + dp_ref[1] + 1.0
    dinv_ref[...] = lax.rsqrt(deg)


def _deg_to_dinv(degp3):
    return pl.pallas_call(
        _k7_body,
        out_shape=jax.ShapeDtypeStruct((_NPAD // 128, 128), jnp.float32),
    )(degp3)


_CB = 25
_NB = _NCH // _CB


def _k7b_body(dv_ref, xw_ref, t_ref, u_ref):
    t = dv_ref[...] * xw_ref[...]
    t_ref[...] = t
    u_ref[...] = dv_ref[...] * t


def _scale_xw(dinv2, xw):
    return pl.pallas_call(
        _k7b_body,
        out_shape=(jax.ShapeDtypeStruct((_N, _D), jnp.float32),
                   jax.ShapeDtypeStruct((_N, _D), jnp.float32)),
    )(dinv2, xw)


def _dyn_splat16(vec16, lane):
    return lax.gather(
        vec16, jnp.full((16, 1), lane, jnp.int32),
        lax.GatherDimensionNumbers(offset_dims=(),
                                   collapsed_slice_dims=(0,),
                                   start_index_map=(0,)),
        slice_sizes=(1,),
        mode=lax.GatherScatterMode.PROMISE_IN_BOUNDS)


def _sc_agg(t, src4, dst4, ew4, zrows):
    @functools.partial(
        pl.kernel,
        out_type=jax.ShapeDtypeStruct((_NC, _NPAD, _D), jnp.float32),
        mesh=_sc_mesh(),
        scratch_types=[
            pltpu.VMEM((_CB, _C), jnp.int32),
            pltpu.VMEM((_CB, _C), jnp.int32),
            pltpu.VMEM((_CB, _C), jnp.float32),
            pltpu.VMEM((_C, _D), jnp.float32),
            pltpu.VMEM_SHARED((_NPAD, _D), jnp.float32),
            pltpu.SemaphoreType.DMA,
        ],
    )
    def k(t_hbm, src_hbm, dst_hbm, ew_hbm, z_hbm, out_hbm,
          sidx, didx, ewv, gb, aggs, sem):
        cid = lax.axis_index("c")
        sid = lax.axis_index("s")
        wid = cid * _NS + sid
        rbase = pl.multiple_of(sid * _RPT, 8)
        pltpu.sync_copy(z_hbm, aggs.at[pl.ds(rbase, _RPT)])
        plsc.subcore_barrier()

        def block(b, carry):
            pltpu.sync_copy(src_hbm.at[wid, b], sidx)
            pltpu.sync_copy(dst_hbm.at[wid, b], didx)
            pltpu.sync_copy(ew_hbm.at[wid, b], ewv)

            def chunk(i, carry1):
                pltpu.async_copy(t_hbm.at[sidx.at[i]], gb, sem).wait()

                def grp(g, carry2):
                    gv = ewv[i, pl.ds(g * 16, 16)]

                    def lane(t2, carry3):
                        r = g * 16 + t2
                        splat = _dyn_splat16(gv, t2)
                        for j in range(_D // 16):
                            sl = pl.ds(j * 16, 16)
                            gb[r, sl] = gb[r, sl] * splat
                        return carry3

                    lax.fori_loop(0, 16, lane, 0)
                    return carry2

                lax.fori_loop(0, _C // 16, grp, 0)
                pltpu.sync_copy(gb, aggs.at[didx.at[i]], add=True)
                return carry1

            lax.fori_loop(0, _CB, chunk, 0)
            return carry

        lax.fori_loop(0, _NB, block, 0)
        plsc.subcore_barrier()
        pltpu.sync_copy(aggs.at[pl.ds(rbase, _RPT)],
                        out_hbm.at[cid, pl.ds(rbase, _RPT)])

    return k(t, src4, dst4, ew4, zrows)


def _k9_body(ap_ref, dv_ref, u_ref, x_ref, g_ref, b_ref, o_ref):
    agg = dv_ref[...] * (ap_ref[0, :_N] + ap_ref[1, :_N]) + u_ref[...]
    mu = jnp.mean(agg, axis=0)
    var = jnp.mean((agg - mu) ** 2, axis=0)
    x1 = jnp.maximum((agg - mu) * lax.rsqrt(var + 1e-5) * g_ref[...]
                     + b_ref[...], 0.0)
    o_ref[...] = x1 + x_ref[...]


def _node_bn(aggp, dinv2, u, x, gamma, beta):
    return pl.pallas_call(
        _k9_body,
        out_shape=jax.ShapeDtypeStruct((_N, _D), jnp.float32),
    )(aggp, dinv2, u, x, gamma, beta)


def kernel(x, edge_index, edge_f, edge_attr, lin1_W, lin1_b, lin2_W, lin2_b,
           linw_W, linw_b, gcn_W, gcn_b, bn_e_gamma, bn_e_beta,
           bn_n_gamma, bn_n_beta):
    src = edge_index[0]
    dst = edge_index[1]
    src3 = src.reshape(_NW, _NCH, _C)
    dst3 = dst.reshape(_NW, _NCH, _C)

    h, xw = _dense_pre(x, lin1_W, gcn_W, lin1_b)
    s = _sc_gather_relu(h, src3, dst3)

    wem = lin2_W[:, :_D]
    wea = lin2_W[:, _D:_D + _DE]
    wef = lin2_W[:, _D + _DE:]
    raw, stats = _edge_mlp(s, edge_attr, edge_f, wem, wea, wef)

    raw2 = raw.reshape(_E // 8, 128)
    ea2 = edge_attr.reshape(_E // 8, 128)
    selb = (jnp.arange(128, dtype=jnp.int32)[:, None] // _DE
            == jnp.arange(8, dtype=jnp.int32)[None, :]).astype(jnp.float32)
    oef2, ew2 = _edge_bn(raw2, stats, bn_e_gamma, bn_e_beta,
                         linw_W, linw_b, ea2, selb)
    out_ef = oef2.reshape(_E, _DE)
    ew3 = ew2.reshape(_NW, _NCH, _C)

    degp = _sc_deg(dst3, ew3)
    dinv80 = _deg_to_dinv(degp.reshape(_NC, _NPAD // 128, 128))
    dinv2 = dinv80.reshape(_NPAD)[:_N].reshape(_N, 1)

    t, u = _scale_xw(dinv2, xw)
    src4 = src.reshape(_NW, _NB, _CB, _C)
    dst4 = dst.reshape(_NW, _NB, _CB, _C)
    ew4 = ew2.reshape(_NW, _NB, _CB, _C)
    zrows = jnp.zeros((_RPT, _D), jnp.float32)
    aggp = _sc_agg(t, src4, dst4, ew4, zrows)
    out_x = _node_bn(aggp, dinv2, u, x, bn_n_gamma, bn_n_beta)
    return (out_x, out_ef)
